# baseline (speedup 1.0000x reference)
import functools

import jax
import jax.numpy as jnp
from jax import lax
from jax.experimental import pallas as pl
from jax.experimental.pallas import tpu as pltpu
from jax.experimental.pallas import tpu_sc as plsc

N_CHANNELS = 4096
N_BINS = 8192
BLOCK_ROWS = 256

SC_ROWS = 512
TC_ROWS = N_CHANNELS - SC_ROWS
SC_BASE = TC_ROWS

NUM_WORKERS = 32
R_PER = SC_ROWS // NUM_WORKERS
CHUNK = 4
NCHUNKS = R_PER // CHUNK
LANES = 16


def _flux(params_ref, energies_ref):
    alpha = params_ref[0]
    norm = params_ref[1]
    p = 1.0 - alpha
    e_low = energies_ref[0:1, :]
    e_high = energies_ref[1:2, :]
    return norm * (jnp.power(e_high, p) - jnp.power(e_low, p)) / p


def _flux_body(params_ref, energies_ref, out_ref):
    out_ref[...] = _flux(params_ref, energies_ref)


def _tc_body(params_ref, energies_ref, tm_ref, out_ref, flux_ref):
    @pl.when(pl.program_id(0) == 0)
    def _():
        flux_ref[...] = _flux(params_ref, energies_ref)

    acc = jax.lax.dot_general(
        tm_ref[...], flux_ref[...],
        (((1,), (1,)), ((), ())),
        preferred_element_type=jnp.float32,
    )
    out_ref[...] = jnp.maximum(acc, 1e-6)


def _sc_body(tm_hbm, flux_hbm, out_hbm, flux_v, rows_v, acc_v, out_v,
             sem0, sem1):
    wid = lax.axis_index("s") * 2 + lax.axis_index("c")
    row0 = SC_BASE + wid * R_PER

    def copy(i, slot, sem):
        return pltpu.make_async_copy(
            tm_hbm.at[pl.ds(row0 + i * CHUNK, CHUNK), :],
            rows_v.at[slot],
            sem,
        )

    copy(0, 0, sem0).start()
    pltpu.sync_copy(flux_hbm, flux_v)

    iota = lax.iota(jnp.int32, LANES)
    for i in range(NCHUNKS):
        slot = i % 2
        sem = sem0 if slot == 0 else sem1
        if i + 1 < NCHUNKS:
            nslot = (i + 1) % 2
            copy(i + 1, nslot, sem1 if nslot else sem0).start()
        copy(i, slot, sem).wait()
        for r in range(CHUNK):
            def step(c, acc, _slot=slot, _r=r):
                a = rows_v[_slot, _r, pl.ds(c * 64, LANES)]
                b = flux_v[pl.ds(c * 64, LANES)]
                acc = acc + a * b
                for j in range(1, 4):
                    a = rows_v[_slot, _r, pl.ds(c * 64 + j * LANES, LANES)]
                    b = flux_v[pl.ds(c * 64 + j * LANES, LANES)]
                    acc = acc + a * b
                return acc

            acc = lax.fori_loop(0, N_BINS // 64, step,
                                jnp.zeros((LANES,), jnp.float32))
            acc_v[i * CHUNK + r] = acc

    tot = jnp.zeros((LANES,), jnp.float32)
    for c in range(LANES):
        tot = tot + plsc.load_gather(
            acc_v, [iota, jnp.full((LANES,), c, jnp.int32)])
    out_v[...] = jnp.maximum(tot, 1e-6)
    pltpu.sync_copy(out_v, out_hbm.at[pl.ds(wid * R_PER, R_PER)])


_sc_gemv = pl.kernel(
    _sc_body,
    out_type=jax.ShapeDtypeStruct((SC_ROWS,), jnp.float32),
    mesh=plsc.VectorSubcoreMesh(core_axis_name="c", subcore_axis_name="s"),
    compiler_params=pltpu.CompilerParams(needs_layout_passes=False),
    scratch_types=[
        pltpu.VMEM((N_BINS,), jnp.float32),
        pltpu.VMEM((2, CHUNK, N_BINS), jnp.float32),
        pltpu.VMEM((LANES, LANES), jnp.float32),
        pltpu.VMEM((R_PER,), jnp.float32),
        pltpu.SemaphoreType.DMA,
        pltpu.SemaphoreType.DMA,
    ],
)


def kernel(parameters, transfer_matrix, energies):
    flux = pl.pallas_call(
        _flux_body,
        in_specs=[
            pl.BlockSpec(memory_space=pltpu.MemorySpace.SMEM),
            pl.BlockSpec(memory_space=pltpu.MemorySpace.VMEM),
        ],
        out_specs=pl.BlockSpec(memory_space=pltpu.MemorySpace.VMEM),
        out_shape=jax.ShapeDtypeStruct((1, N_BINS), jnp.float32),
    )(parameters, energies)

    sc_out = _sc_gemv(transfer_matrix, flux.reshape(N_BINS))

    tc_out = pl.pallas_call(
        _tc_body,
        grid=(TC_ROWS // BLOCK_ROWS,),
        in_specs=[
            pl.BlockSpec(memory_space=pltpu.MemorySpace.SMEM),
            pl.BlockSpec((2, N_BINS), lambda i: (0, 0)),
            pl.BlockSpec((BLOCK_ROWS, N_BINS), lambda i: (i, 0)),
        ],
        out_specs=pl.BlockSpec((BLOCK_ROWS, 1), lambda i: (i, 0)),
        out_shape=jax.ShapeDtypeStruct((TC_ROWS, 1), jnp.float32),
        scratch_shapes=[pltpu.MemorySpace.VMEM((1, N_BINS), jnp.float32)],
    )(parameters, energies, transfer_matrix)

    return jnp.concatenate([tc_out.reshape(TC_ROWS), sc_out])


# speedup vs baseline: 1.0045x; 1.0045x over previous
import functools

import jax
import jax.numpy as jnp
from jax import lax
from jax.experimental import pallas as pl
from jax.experimental.pallas import tpu as pltpu
from jax.experimental.pallas import tpu_sc as plsc

N_CHANNELS = 4096
N_BINS = 8192
BLOCK_ROWS = 256

SC_ROWS = 512
TC_ROWS = N_CHANNELS - SC_ROWS
SC_BASE = TC_ROWS

NUM_WORKERS = 32
R_PER = SC_ROWS // NUM_WORKERS
CHUNK = 4
NCHUNKS = R_PER // CHUNK
LANES = 16


def _flux(params_ref, energies_ref):
    alpha = params_ref[0]
    norm = params_ref[1]
    p = 1.0 - alpha
    e_low = energies_ref[0:1, :]
    e_high = energies_ref[1:2, :]
    return norm * (jnp.power(e_high, p) - jnp.power(e_low, p)) / p


def _flux_body(params_ref, energies_ref, out_ref):
    out_ref[...] = _flux(params_ref, energies_ref)


def _tc_body(params_ref, energies_ref, tm_ref, out_ref, flux_ref):
    @pl.when(pl.program_id(0) == 0)
    def _():
        flux_ref[...] = _flux(params_ref, energies_ref)

    acc = jax.lax.dot_general(
        tm_ref[...], flux_ref[...],
        (((1,), (1,)), ((), ())),
        preferred_element_type=jnp.float32,
    )
    out_ref[...] = jnp.maximum(acc, 1e-6)


def _sc_body(tm_hbm, flux_hbm, out_hbm, flux_v, rows_v, acc_v, out_v,
             sem0, sem1):
    wid = lax.axis_index("s") * 2 + lax.axis_index("c")
    row0 = SC_BASE + wid * R_PER

    def copy(i, slot, sem):
        return pltpu.make_async_copy(
            tm_hbm.at[pl.ds(row0 + i * CHUNK, CHUNK), :],
            rows_v.at[slot],
            sem,
        )

    copy(0, 0, sem0).start()
    pltpu.sync_copy(flux_hbm, flux_v)

    iota = lax.iota(jnp.int32, LANES)
    for i in range(NCHUNKS):
        slot = i % 2
        sem = sem0 if slot == 0 else sem1
        if i + 1 < NCHUNKS:
            nslot = (i + 1) % 2
            copy(i + 1, nslot, sem1 if nslot else sem0).start()
        copy(i, slot, sem).wait()
        for r in range(CHUNK):
            def step(c, acc, _slot=slot, _r=r):
                a = rows_v[_slot, _r, pl.ds(c * 64, LANES)]
                b = flux_v[pl.ds(c * 64, LANES)]
                acc = acc + a * b
                for j in range(1, 4):
                    a = rows_v[_slot, _r, pl.ds(c * 64 + j * LANES, LANES)]
                    b = flux_v[pl.ds(c * 64 + j * LANES, LANES)]
                    acc = acc + a * b
                return acc

            acc = lax.fori_loop(0, N_BINS // 64, step,
                                jnp.zeros((LANES,), jnp.float32))
            acc_v[i * CHUNK + r] = acc

    tot = jnp.zeros((LANES,), jnp.float32)
    for c in range(LANES):
        tot = tot + plsc.load_gather(
            acc_v, [iota, jnp.full((LANES,), c, jnp.int32)])
    out_v[...] = jnp.maximum(tot, 1e-6)
    pltpu.sync_copy(out_v, out_hbm.at[pl.ds(wid * R_PER, R_PER)])


_sc_gemv = pl.kernel(
    _sc_body,
    out_type=jax.ShapeDtypeStruct((SC_ROWS,), jnp.float32),
    mesh=plsc.VectorSubcoreMesh(core_axis_name="c", subcore_axis_name="s"),
    compiler_params=pltpu.CompilerParams(needs_layout_passes=False),
    cost_estimate=pl.CostEstimate(
        flops=2 * SC_ROWS * N_BINS,
        transcendentals=0,
        bytes_accessed=SC_ROWS * N_BINS * 4 + N_BINS * 4 + SC_ROWS * 4,
    ),
    scratch_types=[
        pltpu.VMEM((N_BINS,), jnp.float32),
        pltpu.VMEM((2, CHUNK, N_BINS), jnp.float32),
        pltpu.VMEM((LANES, LANES), jnp.float32),
        pltpu.VMEM((R_PER,), jnp.float32),
        pltpu.SemaphoreType.DMA,
        pltpu.SemaphoreType.DMA,
    ],
)


def kernel(parameters, transfer_matrix, energies):
    flux = pl.pallas_call(
        _flux_body,
        in_specs=[
            pl.BlockSpec(memory_space=pltpu.MemorySpace.SMEM),
            pl.BlockSpec(memory_space=pltpu.MemorySpace.VMEM),
        ],
        out_specs=pl.BlockSpec(memory_space=pltpu.MemorySpace.VMEM),
        out_shape=jax.ShapeDtypeStruct((1, N_BINS), jnp.float32),
    )(parameters, energies)

    sc_out = _sc_gemv(transfer_matrix, flux.reshape(N_BINS))

    tc_out = pl.pallas_call(
        _tc_body,
        grid=(TC_ROWS // BLOCK_ROWS,),
        in_specs=[
            pl.BlockSpec(memory_space=pltpu.MemorySpace.SMEM),
            pl.BlockSpec((2, N_BINS), lambda i: (0, 0)),
            pl.BlockSpec((BLOCK_ROWS, N_BINS), lambda i: (i, 0)),
        ],
        out_specs=pl.BlockSpec((BLOCK_ROWS, 1), lambda i: (i, 0)),
        out_shape=jax.ShapeDtypeStruct((TC_ROWS, 1), jnp.float32),
        scratch_shapes=[pltpu.MemorySpace.VMEM((1, N_BINS), jnp.float32)],
        cost_estimate=pl.CostEstimate(
            flops=2 * TC_ROWS * N_BINS,
            transcendentals=2 * N_BINS,
            bytes_accessed=TC_ROWS * N_BINS * 4,
        ),
    )(parameters, energies, transfer_matrix)

    return jnp.concatenate([tc_out.reshape(TC_ROWS), sc_out])


# speedup vs baseline: 1.3650x; 1.3589x over previous
import jax
import jax.numpy as jnp
from jax.experimental import pallas as pl
from jax.experimental.pallas import tpu as pltpu

N_CHANNELS = 4096
N_BINS = 8192
BLOCK_ROWS = 256
HALF = N_CHANNELS // 2
NSTEP = HALF // BLOCK_ROWS


def _flux(params_ref, energies_ref):
    alpha = params_ref[0]
    norm = params_ref[1]
    p = 1.0 - alpha
    e_low = energies_ref[0:1, :]
    e_high = energies_ref[1:2, :]
    return norm * (jnp.power(e_high, p) - jnp.power(e_low, p)) / p


def _body(params_ref, energies_ref, top_ref, bot_ref, out_ref, flux_ref):
    @pl.when(pl.program_id(0) == 0)
    def _():
        flux_ref[...] = _flux(params_ref, energies_ref)

    flux = flux_ref[...]
    acc_t = jnp.sum(top_ref[...] * flux, axis=1, keepdims=True)
    acc_b = jnp.sum(bot_ref[...] * flux, axis=1, keepdims=True)
    out_ref[0] = jnp.maximum(acc_t, 1e-6)
    out_ref[1] = jnp.maximum(acc_b, 1e-6)


def kernel(parameters, transfer_matrix, energies):
    out = pl.pallas_call(
        _body,
        grid=(NSTEP,),
        in_specs=[
            pl.BlockSpec(memory_space=pltpu.MemorySpace.SMEM),
            pl.BlockSpec((2, N_BINS), lambda i: (0, 0)),
            pl.BlockSpec((BLOCK_ROWS, N_BINS), lambda i: (i, 0)),
            pl.BlockSpec((BLOCK_ROWS, N_BINS), lambda i: (i + NSTEP, 0)),
        ],
        out_specs=pl.BlockSpec((2, BLOCK_ROWS, 1), lambda i: (0, i, 0)),
        out_shape=jax.ShapeDtypeStruct((2, HALF, 1), jnp.float32),
        scratch_shapes=[pltpu.MemorySpace.VMEM((1, N_BINS), jnp.float32)],
    )(parameters, energies, transfer_matrix, transfer_matrix)
    return out.reshape(N_CHANNELS)


# speedup vs baseline: 1.4288x; 1.0467x over previous
import jax
import jax.numpy as jnp
from jax.experimental import pallas as pl
from jax.experimental.pallas import tpu as pltpu

N_CHANNELS = 4096
N_BINS = 8192
BLOCK_ROWS = 256


def _body(params_ref, energies_ref, tm_ref, out_ref, flux_ref):
    @pl.when(pl.program_id(0) == 0)
    def _():
        alpha = params_ref[0]
        norm = params_ref[1]
        p = 1.0 - alpha
        e_low = energies_ref[0:1, :]
        e_high = energies_ref[1:2, :]
        flux_ref[...] = norm * (jnp.power(e_high, p) - jnp.power(e_low, p)) / p

    acc = jax.lax.dot_general(
        tm_ref[...], flux_ref[...],
        (((1,), (1,)), ((), ())),
        preferred_element_type=jnp.float32,
    )
    out_ref[...] = jnp.maximum(acc, 1e-6)


def kernel(parameters, transfer_matrix, energies):
    out = pl.pallas_call(
        _body,
        grid=(N_CHANNELS // BLOCK_ROWS,),
        in_specs=[
            pl.BlockSpec(memory_space=pltpu.MemorySpace.SMEM),
            pl.BlockSpec((2, N_BINS), lambda i: (0, 0)),
            pl.BlockSpec((BLOCK_ROWS, N_BINS), lambda i: (i, 0)),
        ],
        out_specs=pl.BlockSpec((BLOCK_ROWS, 1), lambda i: (i, 0)),
        out_shape=jax.ShapeDtypeStruct((N_CHANNELS, 1), jnp.float32),
        scratch_shapes=[pltpu.MemorySpace.VMEM((1, N_BINS), jnp.float32)],
    )(parameters, energies, transfer_matrix)
    return out.reshape(N_CHANNELS)
